# baseline (speedup 1.0000x reference)
import functools

import jax
import jax.numpy as jnp
from jax import lax
from jax.experimental import pallas as pl
from jax.experimental.pallas import tpu as pltpu
from jax.experimental.pallas import tpu_sc as plsc

NUM_CORES = 2
NUM_SUBCORES = 16
NUM_WORKERS = NUM_CORES * NUM_SUBCORES

GATHER_ROWS = 128


def _make_kernel(total, dim):
    assert total % (NUM_WORKERS * GATHER_ROWS) == 0
    per_worker = total // NUM_WORKERS
    gathers = per_worker // GATHER_ROWS

    mesh = plsc.VectorSubcoreMesh(
        core_axis_name="c", subcore_axis_name="s"
    )

    @functools.partial(
        pl.kernel,
        out_type=jax.ShapeDtypeStruct((total, dim), jnp.float32),
        mesh=mesh,
        compiler_params=pltpu.CompilerParams(use_tc_tiling_on_sc=False),
        scratch_types=[
            pltpu.VMEM((gathers, GATHER_ROWS), jnp.int32),
            pltpu.VMEM((GATHER_ROWS, dim), jnp.float32),
            pltpu.SemaphoreType.DMA,
        ],
    )
    def gather_kernel(idx_hbm, table_hbm, out_hbm, idx_v, rows_v, sem):
        wid = lax.axis_index("s") * NUM_CORES + lax.axis_index("c")
        pltpu.sync_copy(idx_hbm.at[wid], idx_v)
        row_base = wid * per_worker

        def body(j, carry):
            pltpu.async_copy(table_hbm.at[idx_v.at[j]], rows_v, sem).wait()
            pltpu.sync_copy(
                rows_v, out_hbm.at[pl.ds(row_base + j * GATHER_ROWS, GATHER_ROWS)]
            )
            return carry

        lax.fori_loop(0, gathers, body, 0)

    return gather_kernel


def kernel(token_ids, weight):
    b, t = token_ids.shape
    total = b * t
    dim = weight.shape[1]
    per_worker = total // NUM_WORKERS
    idx3d = token_ids.reshape(
        NUM_WORKERS, per_worker // GATHER_ROWS, GATHER_ROWS
    ).astype(jnp.int32)
    out = _make_kernel(total, dim)(idx3d, weight)
    return out.reshape(b, t, dim)


# speedup vs baseline: 1.0423x; 1.0423x over previous
import functools

import jax
import jax.numpy as jnp
from jax import lax
from jax.experimental import pallas as pl
from jax.experimental.pallas import tpu as pltpu
from jax.experimental.pallas import tpu_sc as plsc

NUM_CORES = 2
NUM_SUBCORES = 16
NUM_WORKERS = NUM_CORES * NUM_SUBCORES

GATHER_ROWS = 128
GATHERS_PER_ROUND = 5
ROUND_ROWS = GATHER_ROWS * GATHERS_PER_ROUND


def _make_kernel(total, dim):
    assert total % (NUM_WORKERS * ROUND_ROWS) == 0
    per_worker = total // NUM_WORKERS
    gathers = per_worker // GATHER_ROWS
    rounds = per_worker // ROUND_ROWS
    assert rounds % 2 == 0 and rounds >= 4

    mesh = plsc.VectorSubcoreMesh(core_axis_name="c", subcore_axis_name="s")

    @functools.partial(
        pl.kernel,
        out_type=jax.ShapeDtypeStruct((total, dim), jnp.float32),
        mesh=mesh,
        compiler_params=pltpu.CompilerParams(use_tc_tiling_on_sc=False),
        scratch_types=[
            pltpu.VMEM((gathers, GATHER_ROWS), jnp.int32),
            pltpu.VMEM((ROUND_ROWS, dim), jnp.float32),
            pltpu.VMEM((ROUND_ROWS, dim), jnp.float32),
            pltpu.SemaphoreType.DMA,
            pltpu.SemaphoreType.DMA,
            pltpu.SemaphoreType.DMA,
            pltpu.SemaphoreType.DMA,
        ],
    )
    def gather_kernel(idx_hbm, table_hbm, out_hbm, idx_v, rows0, rows1,
                      g0, g1, w0, w1):
        wid = lax.axis_index("s") * NUM_CORES + lax.axis_index("c")
        pltpu.sync_copy(idx_hbm.at[wid], idx_v)
        row_base = wid * per_worker

        def fire_round(j, buf, sem):
            for g in range(GATHERS_PER_ROUND):
                pltpu.async_copy(
                    table_hbm.at[idx_v.at[j * GATHERS_PER_ROUND + g]],
                    buf.at[pl.ds(g * GATHER_ROWS, GATHER_ROWS)],
                    sem,
                )

        def drain_round(j, buf, sem):
            for g in range(GATHERS_PER_ROUND):
                pltpu.make_async_copy(
                    table_hbm.at[idx_v.at[j * GATHERS_PER_ROUND + g]],
                    buf.at[pl.ds(g * GATHER_ROWS, GATHER_ROWS)],
                    sem,
                ).wait()

        def fire_wb(j, buf, sem):
            pltpu.async_copy(
                buf, out_hbm.at[pl.ds(row_base + j * ROUND_ROWS, ROUND_ROWS)], sem
            )

        def drain_wb(j, buf, sem):
            pltpu.make_async_copy(
                buf, out_hbm.at[pl.ds(row_base + j * ROUND_ROWS, ROUND_ROWS)], sem
            ).wait()

        fire_round(0, rows0, g0)

        def body(i, carry):
            r0 = 2 * i
            r1 = r0 + 1

            drain_round(r0, rows0, g0)
            fire_wb(r0, rows0, w0)
            @pl.when(i > 0)
            def _():
                drain_wb(r0 - 1, rows1, w1)

            fire_round(r1, rows1, g1)

            drain_round(r1, rows1, g1)
            fire_wb(r1, rows1, w1)
            @pl.when(i < rounds // 2 - 1)
            def _():
                drain_wb(r0, rows0, w0)
                fire_round(r1 + 1, rows0, g0)

            return carry

        lax.fori_loop(0, rounds // 2, body, 0)

        drain_wb(rounds - 2, rows0, w0)
        drain_wb(rounds - 1, rows1, w1)

    return gather_kernel


def kernel(token_ids, weight):
    b, t = token_ids.shape
    total = b * t
    dim = weight.shape[1]
    per_worker = total // NUM_WORKERS
    idx3d = token_ids.reshape(
        NUM_WORKERS, per_worker // GATHER_ROWS, GATHER_ROWS
    ).astype(jnp.int32)
    out = _make_kernel(total, dim)(idx3d, weight)
    return out.reshape(b, t, dim)
